# baseline (speedup 1.0000x reference)
import functools

import jax
import jax.numpy as jnp
from jax import lax
from jax.experimental import pallas as pl
from jax.experimental.pallas import tpu as pltpu
from jax.experimental.pallas import tpu_sc as plsc

N_VOX = 50000
K_VOL = 27
PAIRS = 25000
C = 64

HALF = N_VOX // 2
TRASH = HALF
ACC_ROWS = HALF + 8
CHUNK = 128
N_SUB = 16
N_PAIRS = K_VOL * PAIRS
N_PAIRS_PAD = 675840
CHUNKS_PER_TILE = N_PAIRS_PAD // CHUNK // N_SUB
SEG = 30
N_SEG = CHUNKS_PER_TILE // SEG
INIT_ROWS_PER_TILE = ACC_ROWS // N_SUB
OUT_FULL_CHUNKS = HALF // CHUNK
OUT_TAIL = HALF - OUT_FULL_CHUNKS * CHUNK

MM_BLOCK = 5000


def _mm_body(x_ref, w_ref, y_ref):
    w = w_ref[0]
    y_ref[:, :C] = jnp.dot(
        x_ref[:, :C], w, preferred_element_type=jnp.float32
    )
    y_ref[:, C:] = jnp.dot(
        x_ref[:, C:], w, preferred_element_type=jnp.float32
    )


def _dense_transform(in2, weights):
    nb = (N_VOX // 2) // MM_BLOCK
    return pl.pallas_call(
        _mm_body,
        grid=(nb, K_VOL),
        in_specs=[
            pl.BlockSpec((MM_BLOCK, 2 * C), lambda j, k: (j, 0)),
            pl.BlockSpec((1, C, C), lambda j, k: (k, 0, 0)),
        ],
        out_specs=pl.BlockSpec((MM_BLOCK, 2 * C), lambda j, k: (k * nb + j, 0)),
        out_shape=jax.ShapeDtypeStruct((K_VOL * N_VOX // 2, 2 * C), jnp.float32),
    )(in2, weights)


def _sc_scatter(y_flat, gidx2, omap2, binit):
    mesh = plsc.VectorSubcoreMesh(core_axis_name="c", subcore_axis_name="s")

    @functools.partial(
        pl.kernel,
        mesh=mesh,
        compiler_params=pltpu.CompilerParams(use_tc_tiling_on_sc=False),
        out_type=jax.ShapeDtypeStruct((N_VOX, C), jnp.float32),
        scratch_types=[
            pltpu.VMEM((SEG, CHUNK), jnp.int32),
            pltpu.VMEM((SEG, CHUNK), jnp.int32),
            pltpu.VMEM((CHUNK,), jnp.int32),
            pltpu.VMEM((CHUNK, C), jnp.float32),
            pltpu.VMEM((CHUNK, C), jnp.float32),
            pltpu.VMEM_SHARED((ACC_ROWS, C), jnp.float32),
            pltpu.SemaphoreType.DMA,
            pltpu.SemaphoreType.DMA,
            pltpu.SemaphoreType.DMA,
        ],
    )
    def body(y_hbm, gidx_hbm, omap_hbm, binit_hbm, out_hbm,
             gidx_seg, omap_seg, idx_v, rows0, rows1, acc,
             sem0, sem1, sem_out):
        cid = lax.axis_index("c")
        sid = lax.axis_index("s")
        row_base = cid * HALF
        rows_b = (rows0, rows1)
        sem_b = (sem0, sem1)

        init_base = sid * INIT_ROWS_PER_TILE
        pltpu.sync_copy(
            binit_hbm.at[pl.ds(init_base, INIT_ROWS_PER_TILE)],
            acc.at[pl.ds(init_base, INIT_ROWS_PER_TILE)],
        )
        plsc.subcore_barrier()

        tile_row0 = sid * CHUNKS_PER_TILE

        def fire(j, b):
            pltpu.make_async_copy(
                y_hbm.at[gidx_seg.at[j]], rows_b[b], sem_b[b]
            ).start()

        def drain_and_scatter(j, b):
            pltpu.make_async_copy(
                y_hbm.at[gidx_seg.at[j]], rows_b[b], sem_b[b]
            ).wait()
            for v in range(CHUNK // 16):
                o = omap_seg[j, pl.ds(v * 16, 16)]
                loc = o - row_base
                ok = (loc >= 0) & (loc < HALF)
                idx_v[pl.ds(v * 16, 16)] = jnp.where(ok, loc, TRASH)
            pltpu.sync_copy(rows_b[b], acc.at[idx_v], add=True)

        def seg_step(s, _):
            srow = tile_row0 + s * SEG
            pltpu.sync_copy(gidx_hbm.at[pl.ds(srow, SEG)], gidx_seg)
            pltpu.sync_copy(omap_hbm.at[pl.ds(srow, SEG)], omap_seg)
            fire(0, 0)

            def pair_step(jj, _):
                for b in (0, 1):
                    j = jj * 2 + b

                    @pl.when(j + 1 < SEG)
                    def _():
                        fire(j + 1, 1 - b)

                    drain_and_scatter(j, b)
                return 0

            lax.fori_loop(0, SEG // 2, pair_step, 0)
            return 0

        lax.fori_loop(0, N_SEG, seg_step, 0)
        plsc.subcore_barrier()

        def out_descs():
            for i in range(OUT_FULL_CHUNKS // N_SUB + 1):
                chunk = i * N_SUB + sid
                off = chunk * CHUNK
                full = pltpu.make_async_copy(
                    acc.at[pl.ds(off, CHUNK)],
                    out_hbm.at[pl.ds(row_base + off, CHUNK)],
                    sem_out,
                )
                tail = pltpu.make_async_copy(
                    acc.at[pl.ds(OUT_FULL_CHUNKS * CHUNK, OUT_TAIL)],
                    out_hbm.at[
                        pl.ds(row_base + OUT_FULL_CHUNKS * CHUNK, OUT_TAIL)
                    ],
                    sem_out,
                )
                yield chunk, full, tail

        for chunk, full, tail in out_descs():
            @pl.when(chunk < OUT_FULL_CHUNKS)
            def _():
                full.start()

            @pl.when(chunk == OUT_FULL_CHUNKS)
            def _():
                tail.start()

        for chunk, full, tail in out_descs():
            @pl.when(chunk < OUT_FULL_CHUNKS)
            def _():
                full.wait()

            @pl.when(chunk == OUT_FULL_CHUNKS)
            def _():
                tail.wait()

    return body(y_flat, gidx2, omap2, binit)


def kernel(in_feats, imap, omap, kernel, bias):
    imap = imap.astype(jnp.int32)
    omap = omap.astype(jnp.int32)

    in2 = in_feats.reshape(N_VOX // 2, 2 * C)
    y128 = _dense_transform(in2, kernel)
    y_flat = y128.reshape(K_VOL * N_VOX, C)

    k_off = (jnp.arange(K_VOL, dtype=jnp.int32) * N_VOX)[:, None]
    gidx = (imap + k_off).reshape(-1)
    pad = N_PAIRS_PAD - N_PAIRS
    gidx = jnp.concatenate([gidx, jnp.zeros((pad,), jnp.int32)])
    omap_flat = jnp.concatenate(
        [omap.reshape(-1), jnp.full((pad,), N_VOX, jnp.int32)]
    )
    gidx2 = gidx.reshape(N_SUB * CHUNKS_PER_TILE, CHUNK)
    omap2 = omap_flat.reshape(N_SUB * CHUNKS_PER_TILE, CHUNK)
    binit = jnp.broadcast_to(bias, (ACC_ROWS, C))
    return _sc_scatter(y_flat, gidx2, omap2, binit)
